# baseline (speedup 1.0000x reference)
import functools

import jax
import jax.numpy as jnp
from jax import lax
from jax.experimental import pallas as pl
from jax.experimental.pallas import tpu as pltpu
from jax.experimental.pallas import tpu_sc as plsc

NC = 2
NS = 16
NW = NC * NS
L = 16
K = 128


def _matmul_kernel(x_ref, w_ref, o_ref):
    o_ref[...] = jnp.dot(x_ref[...], w_ref[...],
                         preferred_element_type=jnp.float32)


def _combine_kernel(p0_ref, p1_ref, b_ref, o_ref):
    o_ref[...] = p0_ref[...] + p1_ref[...] + b_ref[...]


def _make_sc_agg(n, n_pad, d, cpt):
    rows_per_tile = n_pad // NS
    mesh = plsc.VectorSubcoreMesh(core_axis_name="c", subcore_axis_name="s")

    @functools.partial(
        pl.kernel,
        out_type=jax.ShapeDtypeStruct((NC, n_pad, d), jnp.float32),
        mesh=mesh,
        scratch_types=[
            pltpu.VMEM((cpt + 2, K), jnp.int32),
            pltpu.VMEM((K, d), jnp.float32),
            pltpu.VMEM((K, d), jnp.float32),
            pltpu.VMEM((K,), jnp.int32),
            pltpu.VMEM((K,), jnp.int32),
            pltpu.VMEM((K,), jnp.int32),
            pltpu.VMEM((K,), jnp.int32),
            pltpu.VMEM((K,), jnp.float32),
            pltpu.VMEM((K,), jnp.float32),
            pltpu.VMEM_SHARED((n_pad, d), jnp.float32),
            pltpu.SemaphoreType.DMA,
            pltpu.SemaphoreType.DMA,
            pltpu.SemaphoreType.DMA,
            pltpu.SemaphoreType.DMA,
            pltpu.SemaphoreType.DMA,
            pltpu.SemaphoreType.DMA,
        ],
    )
    def sc_agg(pre_hbm, packed_hbm, ws_hbm, out_hbm,
               pk_v, rows0, rows1, si0, si1, di0, di1, wb0, wb1, acc_sh,
               g0, g1, s0, s1, w0, w1):
        c = lax.axis_index("c")
        s = lax.axis_index("s")
        wid = c * NS + s
        rows = (rows0, rows1)
        sidx = (si0, si1)
        didx = (di0, di1)
        wbuf = (wb0, wb1)
        gsem = (g0, g1)
        ssem = (s0, s1)
        wsem = (w0, w1)

        with jax.named_scope("stage"):
            pltpu.sync_copy(packed_hbm.at[wid], pk_v)

        with jax.named_scope("zero"):
            zero_v = jnp.zeros((L,), jnp.float32)
            for r in range(K):
                for g in range(d // L):
                    rows0[r, pl.ds(g * L, L)] = zero_v
            sl_base = s * rows_per_tile
            for r0 in range(0, rows_per_tile, K):
                rr = min(K, rows_per_tile - r0)
                pltpu.sync_copy(rows0.at[pl.ds(0, rr)],
                                acc_sh.at[pl.ds(sl_base + r0, rr)])
        with jax.named_scope("bar1"):
            plsc.subcore_barrier()

        def unpack(ci, j):
            for g in range(K // L):
                cs = pl.ds(g * L, L)
                p = pk_v[ci, cs]
                sidx[j][cs] = jnp.bitwise_and(p, 0xFFFF)
                didx[j][cs] = jnp.right_shift(p, 16)

        def multiply(buf, wv):
            def egroup_body(eg, _):
                wg = wv[pl.ds(eg * L, L)]
                for t in range(L):
                    wbc = jnp.full((L,), wg[t], dtype=jnp.float32)
                    e = eg * L + t
                    for g in range(d // L):
                        cs = pl.ds(g * L, L)
                        buf[e, cs] = buf[e, cs] * wbc
                return 0

            lax.fori_loop(0, K // L, egroup_body, 0)

        def step(ci, b):
            ob = 1 - b
            with jax.named_scope("gwait"):
                pltpu.make_async_copy(
                    pre_hbm.at[sidx[b]], rows[b], gsem[b]).wait()
                pltpu.make_async_copy(
                    ws_hbm.at[wid, ci], wbuf[b], wsem[b]).wait()
            with jax.named_scope("mult"):
                multiply(rows[b], wbuf[b])
            with jax.named_scope("swait"):
                pltpu.make_async_copy(
                    rows[ob], acc_sh.at[didx[ob]], ssem[ob]).wait()
            with jax.named_scope("unpk"):
                unpack(ci + 1, ob)
            with jax.named_scope("gissue"):
                pltpu.async_copy(pre_hbm.at[sidx[ob]], rows[ob], gsem[ob])
            with jax.named_scope("wissue"):
                pltpu.async_copy(ws_hbm.at[wid, ci + 2], wbuf[b], wsem[b])
            with jax.named_scope("sissue"):
                pltpu.async_copy(rows[b], acc_sh.at[didx[b]], ssem[b],
                                 add=True)

        unpack(0, 0)
        pltpu.async_copy(pre_hbm.at[si0], rows0, g0)
        pltpu.async_copy(ws_hbm.at[wid, 0], wb0, w0)
        pltpu.async_copy(ws_hbm.at[wid, 1], wb1, w1)

        pltpu.make_async_copy(pre_hbm.at[si0], rows0, g0).wait()
        pltpu.make_async_copy(ws_hbm.at[wid, 0], wb0, w0).wait()
        multiply(rows0, wb0)
        unpack(1, 1)
        pltpu.async_copy(pre_hbm.at[si1], rows1, g1)
        pltpu.async_copy(ws_hbm.at[wid, 2], wb0, w0)
        pltpu.async_copy(rows0, acc_sh.at[di0], s0, add=True)

        def outer(t, _):
            step(2 * t + 1, 1)
            step(2 * t + 2, 0)
            return 0

        lax.fori_loop(0, (cpt - 1) // 2, outer, 0)

        with jax.named_scope("drain"):
            pltpu.make_async_copy(rows0, acc_sh.at[di0], s0).wait()
            pltpu.make_async_copy(pre_hbm.at[si1], rows1, g1).wait()
            pltpu.make_async_copy(ws_hbm.at[wid, 0], wb0, w0).wait()
            pltpu.make_async_copy(ws_hbm.at[wid, 1], wb1, w1).wait()
        with jax.named_scope("bar2"):
            plsc.subcore_barrier()

        with jax.named_scope("outcp"):
            sl = pl.ds(sl_base, rows_per_tile)
            pltpu.sync_copy(acc_sh.at[sl], out_hbm.at[c, sl])

    return sc_agg


def kernel(x, edge_index, edge_weight, W0, b):
    n, d_in = x.shape
    d_out = W0.shape[1]
    e = edge_index.shape[1]

    rb = 400
    grid = n // rb
    pre_sup = pl.pallas_call(
        _matmul_kernel,
        grid=(grid,),
        in_specs=[
            pl.BlockSpec((rb, d_in), lambda i: (i, 0)),
            pl.BlockSpec((d_in, d_out), lambda i: (0, 0)),
        ],
        out_specs=pl.BlockSpec((rb, d_out), lambda i: (i, 0)),
        out_shape=jax.ShapeDtypeStruct((n, d_out), jnp.float32),
    )(x, W0)

    per_tile = -(-e // NW)
    cpt = -(-per_tile // K)
    cpt = cpt + 1 - (cpt % 2)
    per_tile = cpt * K
    e_pad = per_tile * NW
    packed = edge_index[0] | (edge_index[1] << 16)
    packed = jnp.full((e_pad,), n << 16, jnp.int32).at[:e].set(packed)
    w = jnp.zeros((e_pad,), jnp.float32).at[:e].set(edge_weight)
    packeds = jnp.full((NW, cpt + 2, K), n << 16, jnp.int32).at[:, :cpt].set(
        packed.reshape(NW, cpt, K))
    ws = jnp.zeros((NW, cpt + 2, K), jnp.float32).at[:, :cpt].set(
        w.reshape(NW, cpt, K))

    n_pad = -(-(n + 1) // (NS * 8)) * (NS * 8)

    parts = _make_sc_agg(n, n_pad, d_out, cpt)(pre_sup, packeds, ws)

    p0 = parts[0, :n]
    p1 = parts[1, :n]
    out = pl.pallas_call(
        _combine_kernel,
        grid=(grid,),
        in_specs=[
            pl.BlockSpec((rb, d_out), lambda i: (i, 0)),
            pl.BlockSpec((rb, d_out), lambda i: (i, 0)),
            pl.BlockSpec((1, d_out), lambda i: (0, 0)),
        ],
        out_specs=pl.BlockSpec((rb, d_out), lambda i: (i, 0)),
        out_shape=jax.ShapeDtypeStruct((n, d_out), jnp.float32),
    )(p0, p1, b)
    return out


# speedup vs baseline: 2.0318x; 2.0318x over previous
import functools

import jax
import jax.numpy as jnp
from jax import lax
from jax.experimental import pallas as pl
from jax.experimental.pallas import tpu as pltpu
from jax.experimental.pallas import tpu_sc as plsc

NC = 2
NS = 16
NW = NC * NS
L = 16
K = 128


def _matmul_kernel(x_ref, w_ref, o_ref):
    o_ref[...] = jnp.dot(x_ref[...], w_ref[...],
                         preferred_element_type=jnp.float32)


def _combine_kernel(p0_ref, p1_ref, b_ref, o_ref):
    o_ref[...] = p0_ref[...] + p1_ref[...] + b_ref[...]


def _make_sc_agg(n, n_pad, d, cpt):
    rows_per_tile = n_pad // NS
    mesh = plsc.VectorSubcoreMesh(core_axis_name="c", subcore_axis_name="s")

    @functools.partial(
        pl.kernel,
        out_type=jax.ShapeDtypeStruct((NC, n_pad, d), jnp.float32),
        mesh=mesh,
        scratch_types=[
            pltpu.VMEM((cpt + 2, K), jnp.int32),
            pltpu.VMEM((K, d), jnp.float32),
            pltpu.VMEM((K, d), jnp.float32),
            pltpu.VMEM((K,), jnp.int32),
            pltpu.VMEM((K,), jnp.int32),
            pltpu.VMEM((K,), jnp.int32),
            pltpu.VMEM((K,), jnp.int32),
            pltpu.VMEM((K,), jnp.float32),
            pltpu.VMEM((K,), jnp.float32),
            pltpu.VMEM_SHARED((n_pad, d), jnp.float32),
            pltpu.SemaphoreType.DMA,
            pltpu.SemaphoreType.DMA,
            pltpu.SemaphoreType.DMA,
            pltpu.SemaphoreType.DMA,
            pltpu.SemaphoreType.DMA,
            pltpu.SemaphoreType.DMA,
        ],
    )
    def sc_agg(pre_hbm, packed_hbm, ws_hbm, out_hbm,
               pk_v, rows0, rows1, si0, si1, di0, di1, wb0, wb1, acc_sh,
               g0, g1, s0, s1, w0, w1):
        c = lax.axis_index("c")
        s = lax.axis_index("s")
        wid = c * NS + s
        rows = (rows0, rows1)
        sidx = (si0, si1)
        didx = (di0, di1)
        wbuf = (wb0, wb1)
        gsem = (g0, g1)
        ssem = (s0, s1)
        wsem = (w0, w1)

        with jax.named_scope("stage"):
            pltpu.sync_copy(packed_hbm.at[wid], pk_v)

        with jax.named_scope("zero"):
            zero_v = jnp.zeros((L,), jnp.float32)
            for r in range(K):
                for g in range(d // L):
                    rows0[r, pl.ds(g * L, L)] = zero_v
            sl_base = s * rows_per_tile
            for r0 in range(0, rows_per_tile, K):
                rr = min(K, rows_per_tile - r0)
                pltpu.sync_copy(rows0.at[pl.ds(0, rr)],
                                acc_sh.at[pl.ds(sl_base + r0, rr)])
        with jax.named_scope("bar1"):
            plsc.subcore_barrier()

        def unpack(ci, j):
            for g in range(K // L):
                cs = pl.ds(g * L, L)
                p = pk_v[ci, cs]
                sidx[j][cs] = jnp.bitwise_and(p, 0xFFFF)
                didx[j][cs] = jnp.right_shift(p, 16)

        def multiply(buf, wv):
            def egroup_body(eg, _):
                wg = wv[pl.ds(eg * L, L)]
                for t in range(L):
                    wbc = jnp.full((L,), wg[t], dtype=jnp.float32)
                    e = eg * L + t
                    for g in range(d // L):
                        cs = pl.ds(g * L, L)
                        buf[e, cs] = buf[e, cs] * wbc
                return 0

            lax.fori_loop(0, K // L, egroup_body, 0)

        def step(ci, b):
            ob = 1 - b
            with jax.named_scope("gwait"):
                pltpu.make_async_copy(
                    pre_hbm.at[sidx[b]], rows[b], gsem[b]).wait()
                pltpu.make_async_copy(
                    ws_hbm.at[wid, ci], wbuf[b], wsem[b]).wait()
            with jax.named_scope("mult"):
                multiply(rows[b], wbuf[b])
            with jax.named_scope("swait"):
                pltpu.make_async_copy(
                    rows[ob], acc_sh.at[didx[ob]], ssem[ob]).wait()
            with jax.named_scope("unpk"):
                unpack(ci + 1, ob)
            with jax.named_scope("gissue"):
                pltpu.async_copy(pre_hbm.at[sidx[ob]], rows[ob], gsem[ob])
            with jax.named_scope("wissue"):
                pltpu.async_copy(ws_hbm.at[wid, ci + 2], wbuf[b], wsem[b])
            with jax.named_scope("sissue"):
                pltpu.async_copy(rows[b], acc_sh.at[didx[b]], ssem[b],
                                 add=True)

        unpack(0, 0)
        pltpu.async_copy(pre_hbm.at[si0], rows0, g0)
        pltpu.async_copy(ws_hbm.at[wid, 0], wb0, w0)
        pltpu.async_copy(ws_hbm.at[wid, 1], wb1, w1)

        pltpu.make_async_copy(pre_hbm.at[si0], rows0, g0).wait()
        pltpu.make_async_copy(ws_hbm.at[wid, 0], wb0, w0).wait()
        multiply(rows0, wb0)
        unpack(1, 1)
        pltpu.async_copy(pre_hbm.at[si1], rows1, g1)
        pltpu.async_copy(ws_hbm.at[wid, 2], wb0, w0)
        pltpu.async_copy(rows0, acc_sh.at[di0], s0, add=True)

        def outer(t, _):
            step(2 * t + 1, 1)
            step(2 * t + 2, 0)
            return 0

        lax.fori_loop(0, (cpt - 1) // 2, outer, 0)

        with jax.named_scope("drain"):
            pltpu.make_async_copy(rows0, acc_sh.at[di0], s0).wait()
            pltpu.make_async_copy(pre_hbm.at[si1], rows1, g1).wait()
            pltpu.make_async_copy(ws_hbm.at[wid, 0], wb0, w0).wait()
            pltpu.make_async_copy(ws_hbm.at[wid, 1], wb1, w1).wait()
        with jax.named_scope("bar2"):
            plsc.subcore_barrier()

        with jax.named_scope("outcp"):
            sl = pl.ds(sl_base, rows_per_tile)
            pltpu.sync_copy(acc_sh.at[sl], out_hbm.at[c, sl])

    return sc_agg


def kernel(x, edge_index, edge_weight, W0, b):
    n, d_in = x.shape
    d_out = W0.shape[1]
    e = edge_index.shape[1]

    rb = 400
    grid = n // rb
    pre_sup = pl.pallas_call(
        _matmul_kernel,
        grid=(grid,),
        in_specs=[
            pl.BlockSpec((rb, d_in), lambda i: (i, 0)),
            pl.BlockSpec((d_in, d_out), lambda i: (0, 0)),
        ],
        out_specs=pl.BlockSpec((rb, d_out), lambda i: (i, 0)),
        out_shape=jax.ShapeDtypeStruct((n, d_out), jnp.float32),
    )(x, W0)

    per_tile = -(-e // NW)
    cpt = -(-per_tile // K)
    cpt = cpt + 1 - (cpt % 2)
    per_tile = cpt * K
    e_pad = per_tile * NW
    ar = jnp.arange(e_pad, dtype=jnp.int32) % n
    pad_packed = ar | (ar << 16)
    packed = edge_index[0] | (edge_index[1] << 16)
    packed = pad_packed.at[:e].set(packed)
    w = jnp.zeros((e_pad,), jnp.float32).at[:e].set(edge_weight)
    ar2 = jnp.arange(NW * (cpt + 2) * K, dtype=jnp.int32) % n
    pad_packed2 = (ar2 | (ar2 << 16)).reshape(NW, cpt + 2, K)
    packeds = pad_packed2.at[:, :cpt].set(packed.reshape(NW, cpt, K))
    ws = jnp.zeros((NW, cpt + 2, K), jnp.float32).at[:, :cpt].set(
        w.reshape(NW, cpt, K))

    n_pad = -(-(n + 1) // (NS * 8)) * (NS * 8)

    parts = _make_sc_agg(n, n_pad, d_out, cpt)(pre_sup, packeds, ws)

    p0 = parts[0, :n]
    p1 = parts[1, :n]
    out = pl.pallas_call(
        _combine_kernel,
        grid=(grid,),
        in_specs=[
            pl.BlockSpec((rb, d_out), lambda i: (i, 0)),
            pl.BlockSpec((rb, d_out), lambda i: (i, 0)),
            pl.BlockSpec((1, d_out), lambda i: (0, 0)),
        ],
        out_specs=pl.BlockSpec((rb, d_out), lambda i: (i, 0)),
        out_shape=jax.ShapeDtypeStruct((n, d_out), jnp.float32),
    )(p0, p1, b)
    return out


# speedup vs baseline: 2.4097x; 1.1860x over previous
import functools

import jax
import jax.numpy as jnp
from jax import lax
from jax.experimental import pallas as pl
from jax.experimental.pallas import tpu as pltpu
from jax.experimental.pallas import tpu_sc as plsc

NC = 2
NS = 16
NW = NC * NS
L = 16
K = 128


def _matmul_kernel(x_ref, w_ref, o_ref):
    o_ref[...] = jnp.dot(x_ref[...], w_ref[...],
                         preferred_element_type=jnp.float32)


def _combine_kernel(p0_ref, p1_ref, b_ref, o_ref):
    o_ref[...] = p0_ref[...] + p1_ref[...] + b_ref[...]


def _make_sc_agg(n, n_pad, d, cpt):
    rows_per_tile = n_pad // NS
    mesh = plsc.VectorSubcoreMesh(core_axis_name="c", subcore_axis_name="s")

    @functools.partial(
        pl.kernel,
        out_type=jax.ShapeDtypeStruct((NC, n_pad, d), jnp.float32),
        mesh=mesh,
        scratch_types=[
            pltpu.VMEM((cpt + 2, K), jnp.int32),
            pltpu.VMEM((K, d), jnp.float32),
            pltpu.VMEM((K, d), jnp.float32),
            pltpu.VMEM((K,), jnp.int32),
            pltpu.VMEM((K,), jnp.int32),
            pltpu.VMEM((K,), jnp.int32),
            pltpu.VMEM((K,), jnp.int32),
            pltpu.VMEM((K,), jnp.float32),
            pltpu.VMEM((K,), jnp.float32),
            pltpu.VMEM_SHARED((n_pad, d), jnp.float32),
            pltpu.SemaphoreType.DMA,
            pltpu.SemaphoreType.DMA,
            pltpu.SemaphoreType.DMA,
            pltpu.SemaphoreType.DMA,
            pltpu.SemaphoreType.DMA,
            pltpu.SemaphoreType.DMA,
        ],
    )
    def sc_agg(pre_hbm, packed_hbm, ws_hbm, out_hbm,
               pk_v, rows0, rows1, si0, si1, di0, di1, wb0, wb1, acc_sh,
               g0, g1, s0, s1, w0, w1):
        c = lax.axis_index("c")
        s = lax.axis_index("s")
        wid = c * NS + s
        rows = (rows0, rows1)
        sidx = (si0, si1)
        didx = (di0, di1)
        wbuf = (wb0, wb1)
        gsem = (g0, g1)
        ssem = (s0, s1)
        wsem = (w0, w1)

        pltpu.sync_copy(packed_hbm.at[wid], pk_v)

        zero_v = jnp.zeros((L,), jnp.float32)
        for r in range(K):
            for g in range(d // L):
                rows0[r, pl.ds(g * L, L)] = zero_v
        sl_base = s * rows_per_tile
        for r0 in range(0, rows_per_tile, K):
            rr = min(K, rows_per_tile - r0)
            pltpu.sync_copy(rows0.at[pl.ds(0, rr)],
                            acc_sh.at[pl.ds(sl_base + r0, rr)])
        plsc.subcore_barrier()

        def unpack(ci, j):
            for g in range(K // L):
                cs = pl.ds(g * L, L)
                p = pk_v[ci, cs]
                sidx[j][cs] = jnp.bitwise_and(p, 0xFFFF)
                didx[j][cs] = jnp.right_shift(p, 16)

        def multiply(buf, wv):
            def egroup_body(eg, _):
                wg = wv[pl.ds(eg * L, L)]
                for t in range(L):
                    wbc = jnp.full((L,), wg[t], dtype=jnp.float32)
                    e = eg * L + t
                    for g in range(d // L):
                        cs = pl.ds(g * L, L)
                        buf[e, cs] = buf[e, cs] * wbc
                return 0

            lax.fori_loop(0, K // L, egroup_body, 0)

        def step(ci, b):
            ob = 1 - b
            pltpu.make_async_copy(
                pre_hbm.at[sidx[b]], rows[b], gsem[b]).wait()
            pltpu.make_async_copy(
                ws_hbm.at[wid, ci], wbuf[b], wsem[b]).wait()
            pltpu.make_async_copy(
                rows[ob], acc_sh.at[didx[ob]], ssem[ob]).wait()
            unpack(ci + 1, ob)
            pltpu.async_copy(pre_hbm.at[sidx[ob]], rows[ob], gsem[ob])
            multiply(rows[b], wbuf[b])
            pltpu.async_copy(ws_hbm.at[wid, ci + 2], wbuf[b], wsem[b])
            pltpu.async_copy(rows[b], acc_sh.at[didx[b]], ssem[b], add=True)

        unpack(0, 0)
        pltpu.async_copy(pre_hbm.at[si0], rows0, g0)
        pltpu.async_copy(ws_hbm.at[wid, 0], wb0, w0)
        pltpu.async_copy(ws_hbm.at[wid, 1], wb1, w1)

        pltpu.make_async_copy(pre_hbm.at[si0], rows0, g0).wait()
        pltpu.make_async_copy(ws_hbm.at[wid, 0], wb0, w0).wait()
        unpack(1, 1)
        pltpu.async_copy(pre_hbm.at[si1], rows1, g1)
        multiply(rows0, wb0)
        pltpu.async_copy(ws_hbm.at[wid, 2], wb0, w0)
        pltpu.async_copy(rows0, acc_sh.at[di0], s0, add=True)

        def outer(t, _):
            step(2 * t + 1, 1)
            step(2 * t + 2, 0)
            return 0

        lax.fori_loop(0, (cpt - 1) // 2, outer, 0)

        pltpu.make_async_copy(rows0, acc_sh.at[di0], s0).wait()
        pltpu.make_async_copy(pre_hbm.at[si1], rows1, g1).wait()
        pltpu.make_async_copy(ws_hbm.at[wid, 0], wb0, w0).wait()
        pltpu.make_async_copy(ws_hbm.at[wid, 1], wb1, w1).wait()
        plsc.subcore_barrier()

        sl = pl.ds(sl_base, rows_per_tile)
        pltpu.sync_copy(acc_sh.at[sl], out_hbm.at[c, sl])

    return sc_agg


def kernel(x, edge_index, edge_weight, W0, b):
    n, d_in = x.shape
    d_out = W0.shape[1]
    e = edge_index.shape[1]

    rb = 400
    grid = n // rb
    pre_sup = pl.pallas_call(
        _matmul_kernel,
        grid=(grid,),
        in_specs=[
            pl.BlockSpec((rb, d_in), lambda i: (i, 0)),
            pl.BlockSpec((d_in, d_out), lambda i: (0, 0)),
        ],
        out_specs=pl.BlockSpec((rb, d_out), lambda i: (i, 0)),
        out_shape=jax.ShapeDtypeStruct((n, d_out), jnp.float32),
    )(x, W0)

    per_tile = -(-e // NW)
    cpt = -(-per_tile // K)
    cpt = cpt + 1 - (cpt % 2)
    per_tile = cpt * K
    e_pad = per_tile * NW
    ar = jnp.arange(e_pad, dtype=jnp.int32) % n
    pad_packed = ar | (ar << 16)
    packed = edge_index[0] | (edge_index[1] << 16)
    packed = pad_packed.at[:e].set(packed)
    w = jnp.zeros((e_pad,), jnp.float32).at[:e].set(edge_weight)
    ar2 = jnp.arange(NW * (cpt + 2) * K, dtype=jnp.int32) % n
    pad_packed2 = (ar2 | (ar2 << 16)).reshape(NW, cpt + 2, K)
    packeds = pad_packed2.at[:, :cpt].set(packed.reshape(NW, cpt, K))
    ws = jnp.zeros((NW, cpt + 2, K), jnp.float32).at[:, :cpt].set(
        w.reshape(NW, cpt, K))

    n_pad = -(-(n + 1) // (NS * 8)) * (NS * 8)

    parts = _make_sc_agg(n, n_pad, d_out, cpt)(pre_sup, packeds, ws)

    p0 = parts[0, :n]
    p1 = parts[1, :n]
    out = pl.pallas_call(
        _combine_kernel,
        grid=(grid,),
        in_specs=[
            pl.BlockSpec((rb, d_out), lambda i: (i, 0)),
            pl.BlockSpec((rb, d_out), lambda i: (i, 0)),
            pl.BlockSpec((1, d_out), lambda i: (0, 0)),
        ],
        out_specs=pl.BlockSpec((rb, d_out), lambda i: (i, 0)),
        out_shape=jax.ShapeDtypeStruct((n, d_out), jnp.float32),
    )(p0, p1, b)
    return out


# speedup vs baseline: 2.4650x; 1.0230x over previous
import functools

import jax
import jax.numpy as jnp
from jax import lax
from jax.experimental import pallas as pl
from jax.experimental.pallas import tpu as pltpu
from jax.experimental.pallas import tpu_sc as plsc

NC = 2
NS = 16
NW = NC * NS
L = 16
K = 128


def _matmul_kernel(x_ref, w_ref, o_ref):
    o_ref[...] = jnp.dot(x_ref[...], w_ref[...],
                         preferred_element_type=jnp.float32)


def _combine_kernel(p0_ref, p1_ref, b_ref, o_ref):
    o_ref[...] = p0_ref[...] + p1_ref[...] + b_ref[...]


def _make_sc_agg(n, n_pad, d, cpt):
    rows_per_tile = n_pad // NS
    mesh = plsc.VectorSubcoreMesh(core_axis_name="c", subcore_axis_name="s")

    @functools.partial(
        pl.kernel,
        out_type=jax.ShapeDtypeStruct((NC, n_pad, d), jnp.float32),
        mesh=mesh,
        scratch_types=[
            pltpu.VMEM((cpt + 2, K), jnp.int32),
            pltpu.VMEM((K, d), jnp.float32),
            pltpu.VMEM((K, d), jnp.float32),
            pltpu.VMEM((K,), jnp.int32),
            pltpu.VMEM((K,), jnp.int32),
            pltpu.VMEM((K,), jnp.int32),
            pltpu.VMEM((K,), jnp.int32),
            pltpu.VMEM((K,), jnp.float32),
            pltpu.VMEM((K,), jnp.float32),
            pltpu.VMEM_SHARED((n_pad, d), jnp.float32),
            pltpu.SemaphoreType.DMA,
            pltpu.SemaphoreType.DMA,
            pltpu.SemaphoreType.DMA,
            pltpu.SemaphoreType.DMA,
            pltpu.SemaphoreType.DMA,
            pltpu.SemaphoreType.DMA,
        ],
    )
    def sc_agg(pre_hbm, packed_hbm, ws_hbm, out_hbm,
               pk_v, rows0, rows1, si0, si1, di0, di1, wb0, wb1, acc_sh,
               g0, g1, s0, s1, w0, w1):
        c = lax.axis_index("c")
        s = lax.axis_index("s")
        wid = c * NS + s
        rows = (rows0, rows1)
        sidx = (si0, si1)
        didx = (di0, di1)
        wbuf = (wb0, wb1)
        gsem = (g0, g1)
        ssem = (s0, s1)
        wsem = (w0, w1)

        pltpu.sync_copy(packed_hbm.at[wid], pk_v)

        zero_v = jnp.zeros((L,), jnp.float32)
        for r in range(K):
            for g in range(d // L):
                rows0[r, pl.ds(g * L, L)] = zero_v
        sl_base = s * rows_per_tile
        for r0 in range(0, rows_per_tile, K):
            rr = min(K, rows_per_tile - r0)
            pltpu.sync_copy(rows0.at[pl.ds(0, rr)],
                            acc_sh.at[pl.ds(sl_base + r0, rr)])
        plsc.subcore_barrier()

        def unpack(ci, j):
            for g in range(K // L):
                cs = pl.ds(g * L, L)
                p = pk_v[ci, cs]
                sidx[j][cs] = jnp.bitwise_and(p, 0xFFFF)
                didx[j][cs] = jnp.right_shift(p, 16)

        def multiply(buf, wv):
            def egroup_body(eg, _):
                wg = wv[pl.ds(eg * L, L)]
                for t in range(L):
                    wbc = jnp.full((L,), wg[t], dtype=jnp.float32)
                    e = eg * L + t
                    for g in range(d // L):
                        cs = pl.ds(g * L, L)
                        buf[e, cs] = buf[e, cs] * wbc
                return 0

            lax.fori_loop(0, K // L, egroup_body, 0)

        def step(ci, b):
            ob = 1 - b
            pltpu.make_async_copy(
                rows[ob], acc_sh.at[didx[ob]], ssem[ob]).wait()
            unpack(ci + 1, ob)
            pltpu.async_copy(pre_hbm.at[sidx[ob]], rows[ob], gsem[ob])
            pltpu.make_async_copy(
                pre_hbm.at[sidx[b]], rows[b], gsem[b]).wait()
            pltpu.make_async_copy(
                ws_hbm.at[wid, ci], wbuf[b], wsem[b]).wait()
            multiply(rows[b], wbuf[b])
            pltpu.async_copy(ws_hbm.at[wid, ci + 2], wbuf[b], wsem[b])
            pltpu.async_copy(rows[b], acc_sh.at[didx[b]], ssem[b], add=True)

        unpack(0, 0)
        pltpu.async_copy(pre_hbm.at[si0], rows0, g0)
        pltpu.async_copy(ws_hbm.at[wid, 0], wb0, w0)
        pltpu.async_copy(ws_hbm.at[wid, 1], wb1, w1)

        pltpu.make_async_copy(pre_hbm.at[si0], rows0, g0).wait()
        pltpu.make_async_copy(ws_hbm.at[wid, 0], wb0, w0).wait()
        unpack(1, 1)
        pltpu.async_copy(pre_hbm.at[si1], rows1, g1)
        multiply(rows0, wb0)
        pltpu.async_copy(ws_hbm.at[wid, 2], wb0, w0)
        pltpu.async_copy(rows0, acc_sh.at[di0], s0, add=True)

        def outer(t, _):
            step(2 * t + 1, 1)
            step(2 * t + 2, 0)
            return 0

        lax.fori_loop(0, (cpt - 1) // 2, outer, 0)

        pltpu.make_async_copy(rows0, acc_sh.at[di0], s0).wait()
        pltpu.make_async_copy(pre_hbm.at[si1], rows1, g1).wait()
        pltpu.make_async_copy(ws_hbm.at[wid, 0], wb0, w0).wait()
        pltpu.make_async_copy(ws_hbm.at[wid, 1], wb1, w1).wait()
        plsc.subcore_barrier()

        sl = pl.ds(sl_base, rows_per_tile)
        pltpu.sync_copy(acc_sh.at[sl], out_hbm.at[c, sl])

    return sc_agg


def kernel(x, edge_index, edge_weight, W0, b):
    n, d_in = x.shape
    d_out = W0.shape[1]
    e = edge_index.shape[1]

    rb = 400
    grid = n // rb
    pre_sup = pl.pallas_call(
        _matmul_kernel,
        grid=(grid,),
        in_specs=[
            pl.BlockSpec((rb, d_in), lambda i: (i, 0)),
            pl.BlockSpec((d_in, d_out), lambda i: (0, 0)),
        ],
        out_specs=pl.BlockSpec((rb, d_out), lambda i: (i, 0)),
        out_shape=jax.ShapeDtypeStruct((n, d_out), jnp.float32),
    )(x, W0)

    per_tile = -(-e // NW)
    cpt = -(-per_tile // K)
    cpt = cpt + 1 - (cpt % 2)
    per_tile = cpt * K
    e_pad = per_tile * NW
    ar = jnp.arange(e_pad, dtype=jnp.int32) % n
    pad_packed = ar | (ar << 16)
    packed = edge_index[0] | (edge_index[1] << 16)
    packed = pad_packed.at[:e].set(packed)
    w = jnp.zeros((e_pad,), jnp.float32).at[:e].set(edge_weight)
    ar2 = jnp.arange(NW * (cpt + 2) * K, dtype=jnp.int32) % n
    pad_packed2 = (ar2 | (ar2 << 16)).reshape(NW, cpt + 2, K)
    packeds = pad_packed2.at[:, :cpt].set(packed.reshape(NW, cpt, K))
    ws = jnp.zeros((NW, cpt + 2, K), jnp.float32).at[:, :cpt].set(
        w.reshape(NW, cpt, K))

    n_pad = -(-(n + 1) // (NS * 8)) * (NS * 8)

    parts = _make_sc_agg(n, n_pad, d_out, cpt)(pre_sup, packeds, ws)

    p0 = parts[0, :n]
    p1 = parts[1, :n]
    out = pl.pallas_call(
        _combine_kernel,
        grid=(grid,),
        in_specs=[
            pl.BlockSpec((rb, d_out), lambda i: (i, 0)),
            pl.BlockSpec((rb, d_out), lambda i: (i, 0)),
            pl.BlockSpec((1, d_out), lambda i: (0, 0)),
        ],
        out_specs=pl.BlockSpec((rb, d_out), lambda i: (i, 0)),
        out_shape=jax.ShapeDtypeStruct((n, d_out), jnp.float32),
    )(p0, p1, b)
    return out
